# baseline (speedup 1.0000x reference)
import functools

import jax
import jax.numpy as jnp
from jax.experimental import pallas as pl

B, T = 1024, 1024
CONV_DIM, MOE_DIM, FF_DIM, E, NUM_CLASSES = 64, 512, 1024, 6, 10

_HI = jax.lax.Precision.HIGHEST

_NW = 4
_WOUT = 132
_KWIN = 272

_R1 = 16
_R2 = 256


def _frontend_kernel(xw_ref, w1band_ref, b1big_ref, mask_ref, w2stk_ref,
                     b2_ref, projwt_ref, projb_ref, routwt_ref, routb_ref,
                     h_ref, g_ref):
    r = _R1
    xw = xw_ref[...].reshape(r * _NW, _KWIN)
    h1 = jnp.dot(xw, w1band_ref[...], precision=_HI) + b1big_ref[...]
    h1 = jnp.maximum(h1, 0.0)
    a = h1.reshape(r * _NW, _WOUT // 2, 2 * CONV_DIM)
    a = a.reshape(r, _NW, _WOUT // 2, 2 * CONV_DIM)
    a = a * mask_ref[...][None]
    p2 = jnp.concatenate([
        a[:, :, 0:64, 0:CONV_DIM],
        a[:, :, 0:64, CONV_DIM:2 * CONV_DIM],
        a[:, :, 1:65, 0:CONV_DIM],
        a[:, :, 1:65, CONV_DIM:2 * CONV_DIM],
        a[:, :, 2:66, 0:CONV_DIM],
    ], axis=3)
    h2 = jnp.dot(p2.reshape(r * 256, 5 * CONV_DIM), w2stk_ref[...],
                 precision=_HI) + b2_ref[...]
    h2 = jnp.maximum(h2, 0.0).reshape(r, 256, CONV_DIM)
    m = jnp.mean(h2, axis=1)
    h = jnp.dot(m, projwt_ref[...], precision=_HI) + projb_ref[...]
    logits = jnp.dot(h, routwt_ref[...], precision=_HI) + routb_ref[...]
    mx = jnp.max(logits, axis=-1, keepdims=True)
    e = jnp.exp(logits - mx)
    p = e / jnp.sum(e, axis=-1, keepdims=True)
    iota = jax.lax.broadcasted_iota(jnp.int32, (r, 8), 1)
    i1 = jnp.argmax(p, axis=-1)[:, None]
    m1 = iota == i1
    p_masked = jnp.where(m1, -1.0, p)
    i2 = jnp.argmax(p_masked, axis=-1)[:, None]
    m2 = iota == i2
    g = jnp.where(m1 | m2, p, 0.0)
    h_ref[...] = h
    g_ref[...] = g


def _moe_kernel(h_ref, g_ref, w1_ref, b1_ref, w2_ref, b2_ref,
                clswt_ref, clsb_ref, y_ref):
    h = h_ref[...]
    g = g_ref[...]
    acc = jnp.zeros((_R2, MOE_DIM), dtype=jnp.float32)
    for j in range(E):
        hid = jnp.dot(h, w1_ref[j], precision=_HI) + b1_ref[j][None, :]
        hid = jnp.maximum(hid, 0.0)
        eo = jnp.dot(hid, w2_ref[j], precision=_HI) + b2_ref[j][None, :]
        acc = acc + g[:, j:j + 1] * eo
    y_ref[...] = jnp.dot(acc, clswt_ref[...], precision=_HI) + clsb_ref[...]


def kernel(x, conv1_w, conv1_b, conv2_w, conv2_b, proj_w, proj_b,
           router_w, router_b, w1, b1, w2, b2, cls_w, cls_b):
    f32 = jnp.float32
    x = x.astype(f32)
    xpad = jnp.pad(x, ((0, 0), (6, 10)))
    xw = jnp.stack([xpad[:, 256 * i:256 * i + _KWIN] for i in range(_NW)],
                   axis=1)
    wt = conv1_w[:, 0, :].T
    band = jnp.zeros((_KWIN, _WOUT, CONV_DIM), dtype=f32)
    for j in range(_WOUT):
        band = jax.lax.dynamic_update_slice(band, wt[:, None, :],
                                            (2 * j, j, 0))
    w1band = band.reshape(_KWIN, _WOUT * CONV_DIM)
    b1big = jnp.tile(conv1_b, _WOUT)[None, :]
    import numpy as _np
    _i = _np.arange(_NW)[:, None, None]
    _j2 = _np.arange(_WOUT // 2)[None, :, None]
    _l = _np.arange(2 * CONV_DIM)[None, None, :]
    _t = 128 * _i - 2 + 2 * _j2 + (_l >= CONV_DIM)
    mask = jnp.asarray(((_t >= 0) & (_t < 512)).astype(_np.float32))
    w2stk = conv2_w.transpose(2, 1, 0).reshape(5 * CONV_DIM, CONV_DIM)
    b2r = conv2_b[None, :]
    projwt = proj_w.T
    projbr = proj_b[None, :]
    routwt = jnp.pad(router_w.T, ((0, 0), (0, 2)))
    routb = jnp.pad(router_b, (0, 2), constant_values=-1e30)[None, :]

    grid1 = B // _R1
    h, g = pl.pallas_call(
        _frontend_kernel,
        grid=(grid1,),
        in_specs=[
            pl.BlockSpec((_R1, _NW, _KWIN), lambda i: (i, 0, 0)),
            pl.BlockSpec((_KWIN, _WOUT * CONV_DIM), lambda i: (0, 0)),
            pl.BlockSpec((1, _WOUT * CONV_DIM), lambda i: (0, 0)),
            pl.BlockSpec((_NW, _WOUT // 2, 2 * CONV_DIM), lambda i: (0, 0, 0)),
            pl.BlockSpec((5 * CONV_DIM, CONV_DIM), lambda i: (0, 0)),
            pl.BlockSpec((1, CONV_DIM), lambda i: (0, 0)),
            pl.BlockSpec((CONV_DIM, MOE_DIM), lambda i: (0, 0)),
            pl.BlockSpec((1, MOE_DIM), lambda i: (0, 0)),
            pl.BlockSpec((MOE_DIM, 8), lambda i: (0, 0)),
            pl.BlockSpec((1, 8), lambda i: (0, 0)),
        ],
        out_specs=[
            pl.BlockSpec((_R1, MOE_DIM), lambda i: (i, 0)),
            pl.BlockSpec((_R1, 8), lambda i: (i, 0)),
        ],
        out_shape=[
            jax.ShapeDtypeStruct((B, MOE_DIM), f32),
            jax.ShapeDtypeStruct((B, 8), f32),
        ],
    )(xw, w1band, b1big, mask, w2stk, b2r, projwt, projbr, routwt, routb)

    grid2 = B // _R2
    y = pl.pallas_call(
        _moe_kernel,
        grid=(grid2,),
        in_specs=[
            pl.BlockSpec((_R2, MOE_DIM), lambda i: (i, 0)),
            pl.BlockSpec((_R2, 8), lambda i: (i, 0)),
            pl.BlockSpec((E, MOE_DIM, FF_DIM), lambda i: (0, 0, 0)),
            pl.BlockSpec((E, FF_DIM), lambda i: (0, 0)),
            pl.BlockSpec((E, FF_DIM, MOE_DIM), lambda i: (0, 0, 0)),
            pl.BlockSpec((E, MOE_DIM), lambda i: (0, 0)),
            pl.BlockSpec((MOE_DIM, NUM_CLASSES), lambda i: (0, 0)),
            pl.BlockSpec((1, NUM_CLASSES), lambda i: (0, 0)),
        ],
        out_specs=pl.BlockSpec((_R2, NUM_CLASSES), lambda i: (i, 0)),
        out_shape=jax.ShapeDtypeStruct((B, NUM_CLASSES), f32),
    )(h, g, w1, b1, w2, b2, cls_w.T, cls_b[None, :])
    return y


# speedup vs baseline: 2.0697x; 2.0697x over previous
import jax
import jax.numpy as jnp
from jax.experimental import pallas as pl

B, T = 1024, 1024
CONV_DIM, MOE_DIM, FF_DIM, E, NUM_CLASSES = 64, 512, 1024, 6, 10

_HI = jax.lax.Precision.HIGHEST
_BF = jnp.bfloat16

_NW = 4
_WOUT = 132
_KWIN = 272

_R1 = 32
_R2 = 256


def _frontend_kernel(xw_ref, w1band_ref, b1big_ref, mask_ref, w2pack_ref,
                     b2big_ref, projwt_ref, projb_ref, routwt_ref, routb_ref,
                     h_ref, g_ref):
    r = _R1
    xw = xw_ref[...].reshape(r * _NW, _KWIN)
    h1 = jnp.dot(xw, w1band_ref[...],
                 preferred_element_type=jnp.float32) + b1big_ref[...]
    h1 = jnp.maximum(h1, 0.0)
    a = h1.reshape(r * _NW, _WOUT // 2, 2 * CONV_DIM)
    a = a * mask_ref[...]
    a = a.astype(_BF)
    a = jnp.concatenate(
        [a, jnp.zeros((r * _NW, 2, 2 * CONV_DIM), dtype=_BF)], axis=1)
    a4 = a.reshape(r * _NW, 17, 4, 2 * CONV_DIM)
    pieces = []
    for delta in range(11):
        q, half = delta // 2, delta % 2
        blk_lo, blk_hi, slot = (0, 16, q) if q <= 3 else (1, 17, q - 4)
        p = a4[:, blk_lo:blk_hi, slot:slot + 1,
               half * CONV_DIM:(half + 1) * CONV_DIM]
        pieces.append(p.reshape(r * _NW, 16, CONV_DIM))
    p2 = jnp.concatenate(pieces, axis=2)
    h2 = jnp.dot(p2.reshape(r * _NW * 16, 11 * CONV_DIM), w2pack_ref[...],
                 preferred_element_type=jnp.float32) + b2big_ref[...]
    h2 = jnp.maximum(h2, 0.0)
    h2s = jnp.sum(h2.reshape(r, 64, 4 * CONV_DIM), axis=1)
    m = (h2s[:, 0:64] + h2s[:, 64:128] + h2s[:, 128:192]
         + h2s[:, 192:256]) * (1.0 / 256.0)
    h = jnp.dot(m.astype(_BF), projwt_ref[...],
                preferred_element_type=jnp.float32) + projb_ref[...]
    logits = jnp.dot(h.astype(_BF), routwt_ref[...],
                     preferred_element_type=jnp.float32) + routb_ref[...]
    mx = jnp.max(logits, axis=-1, keepdims=True)
    e = jnp.exp(logits - mx)
    p = e / jnp.sum(e, axis=-1, keepdims=True)
    iota = jax.lax.broadcasted_iota(jnp.int32, (r, 8), 1)
    i1 = jnp.argmax(p, axis=-1)[:, None]
    m1 = iota == i1
    p_masked = jnp.where(m1, -1.0, p)
    i2 = jnp.argmax(p_masked, axis=-1)[:, None]
    m2 = iota == i2
    g = jnp.where(m1 | m2, p, 0.0)
    h_ref[...] = h
    g_ref[...] = g


def _moe_kernel(h_ref, g_ref, w1_ref, b1_ref, w2_ref, b2_ref,
                clswt_ref, clsb_ref, y_ref):
    h = h_ref[...].astype(_BF)
    g = g_ref[...]
    acc = jnp.zeros((_R2, MOE_DIM), dtype=jnp.float32)
    for j in range(E):
        hid = jnp.dot(h, w1_ref[j],
                      preferred_element_type=jnp.float32) + b1_ref[j][None, :]
        hid = jnp.maximum(hid, 0.0)
        eo = jnp.dot(hid.astype(_BF), w2_ref[j],
                     preferred_element_type=jnp.float32) + b2_ref[j][None, :]
        acc = acc + g[:, j:j + 1] * eo
    y_ref[...] = jnp.dot(acc.astype(_BF), clswt_ref[...],
                         preferred_element_type=jnp.float32) + clsb_ref[...]


def kernel(x, conv1_w, conv1_b, conv2_w, conv2_b, proj_w, proj_b,
           router_w, router_b, w1, b1, w2, b2, cls_w, cls_b):
    f32 = jnp.float32
    x = x.astype(f32)
    xpad = jnp.pad(x, ((0, 0), (6, 10)))
    xw = jnp.stack([xpad[:, 256 * i:256 * i + _KWIN] for i in range(_NW)],
                   axis=1).astype(_BF)
    wt = conv1_w[:, 0, :].T
    band = jnp.zeros((_KWIN, _WOUT, CONV_DIM), dtype=f32)
    for j in range(_WOUT):
        band = jax.lax.dynamic_update_slice(band, wt[:, None, :],
                                            (2 * j, j, 0))
    w1band = band.reshape(_KWIN, _WOUT * CONV_DIM).astype(_BF)
    b1big = jnp.tile(conv1_b, _WOUT)[None, :]
    import numpy as _np
    _i = _np.arange(_NW)[:, None, None]
    _j2 = _np.arange(_WOUT // 2)[None, :, None]
    _l = _np.arange(2 * CONV_DIM)[None, None, :]
    _t = 128 * _i - 2 + 2 * _j2 + (_l >= CONV_DIM)
    maskf = jnp.asarray(((_t >= 0) & (_t < 512)).astype(_np.float32))
    mask = maskf.reshape(_NW, _WOUT // 2, 2 * CONV_DIM)
    mask = jnp.tile(mask, (_R1, 1, 1)).reshape(_R1 * _NW, _WOUT // 2,
                                               2 * CONV_DIM)
    w2pack = jnp.zeros((11 * CONV_DIM, 4 * CONV_DIM), dtype=f32)
    for delta in range(11):
        for rp in range(4):
            k = delta - 2 * rp
            if 0 <= k < 5:
                w2pack = jax.lax.dynamic_update_slice(
                    w2pack, conv2_w[:, :, k].T,
                    (delta * CONV_DIM, rp * CONV_DIM))
    w2pack = w2pack.astype(_BF)
    b2big = jnp.tile(conv2_b, 4)[None, :]
    projwt = proj_w.T.astype(_BF)
    projbr = proj_b[None, :]
    routwt = jnp.pad(router_w.T, ((0, 0), (0, 2))).astype(_BF)
    routb = jnp.pad(router_b, (0, 2), constant_values=-1e30)[None, :]

    grid1 = B // _R1
    h, g = pl.pallas_call(
        _frontend_kernel,
        grid=(grid1,),
        in_specs=[
            pl.BlockSpec((_R1, _NW, _KWIN), lambda i: (i, 0, 0)),
            pl.BlockSpec((_KWIN, _WOUT * CONV_DIM), lambda i: (0, 0)),
            pl.BlockSpec((1, _WOUT * CONV_DIM), lambda i: (0, 0)),
            pl.BlockSpec((_R1 * _NW, _WOUT // 2, 2 * CONV_DIM),
                         lambda i: (0, 0, 0)),
            pl.BlockSpec((11 * CONV_DIM, 4 * CONV_DIM), lambda i: (0, 0)),
            pl.BlockSpec((1, 4 * CONV_DIM), lambda i: (0, 0)),
            pl.BlockSpec((CONV_DIM, MOE_DIM), lambda i: (0, 0)),
            pl.BlockSpec((1, MOE_DIM), lambda i: (0, 0)),
            pl.BlockSpec((MOE_DIM, 8), lambda i: (0, 0)),
            pl.BlockSpec((1, 8), lambda i: (0, 0)),
        ],
        out_specs=[
            pl.BlockSpec((_R1, MOE_DIM), lambda i: (i, 0)),
            pl.BlockSpec((_R1, 8), lambda i: (i, 0)),
        ],
        out_shape=[
            jax.ShapeDtypeStruct((B, MOE_DIM), f32),
            jax.ShapeDtypeStruct((B, 8), f32),
        ],
    )(xw, w1band, b1big, mask, w2pack, b2big, projwt, projbr, routwt, routb)

    grid2 = B // _R2
    y = pl.pallas_call(
        _moe_kernel,
        grid=(grid2,),
        in_specs=[
            pl.BlockSpec((_R2, MOE_DIM), lambda i: (i, 0)),
            pl.BlockSpec((_R2, 8), lambda i: (i, 0)),
            pl.BlockSpec((E, MOE_DIM, FF_DIM), lambda i: (0, 0, 0)),
            pl.BlockSpec((E, FF_DIM), lambda i: (0, 0)),
            pl.BlockSpec((E, FF_DIM, MOE_DIM), lambda i: (0, 0, 0)),
            pl.BlockSpec((E, MOE_DIM), lambda i: (0, 0)),
            pl.BlockSpec((MOE_DIM, NUM_CLASSES), lambda i: (0, 0)),
            pl.BlockSpec((1, NUM_CLASSES), lambda i: (0, 0)),
        ],
        out_specs=pl.BlockSpec((_R2, NUM_CLASSES), lambda i: (i, 0)),
        out_shape=jax.ShapeDtypeStruct((B, NUM_CLASSES), f32),
    )(h, g, w1.astype(_BF), b1, w2.astype(_BF), b2,
      cls_w.T.astype(_BF), cls_b[None, :])
    return y


# speedup vs baseline: 2.6560x; 1.2833x over previous
import jax
import jax.numpy as jnp
from jax.experimental import pallas as pl

B, T = 1024, 1024
CONV_DIM, MOE_DIM, FF_DIM, E, NUM_CLASSES = 64, 512, 1024, 6, 10

_HI = jax.lax.Precision.HIGHEST
_BF = jnp.bfloat16

_NW = 4
_WOUT = 136
_KWIN = 280

_R1 = 32
_R2 = 256


def _frontend_kernel(xw_ref, w1band_ref, b1big_ref, mask_ref, w2a_ref,
                     w2b_ref, b2big_ref, projwt_ref, projb_ref, routwt_ref,
                     routb_ref, h_ref, g_ref):
    r = _R1
    xw = xw_ref[...].reshape(r * _NW, _KWIN)
    h1 = jnp.dot(xw, w1band_ref[...],
                 preferred_element_type=jnp.float32) + b1big_ref[...]
    h1 = jnp.maximum(h1, 0.0)
    a = h1.reshape(r, _NW, _WOUT // 8, 8 * CONV_DIM)
    a = (a * mask_ref[...][None]).astype(_BF)
    lhs1 = a[:, :, 0:16, :].reshape(r * _NW * 16, 8 * CONV_DIM)
    lhs2 = a[:, :, 1:17, 0:4 * CONV_DIM].reshape(r * _NW * 16, 4 * CONV_DIM)
    h2 = (jnp.dot(lhs1, w2a_ref[...], preferred_element_type=jnp.float32)
          + jnp.dot(lhs2, w2b_ref[...], preferred_element_type=jnp.float32)
          + b2big_ref[...])
    h2 = jnp.maximum(h2, 0.0)
    h2s = jnp.sum(h2.reshape(r, 64, 4 * CONV_DIM), axis=1)
    m = (h2s[:, 0:64] + h2s[:, 64:128] + h2s[:, 128:192]
         + h2s[:, 192:256]) * (1.0 / 256.0)
    h = jnp.dot(m.astype(_BF), projwt_ref[...],
                preferred_element_type=jnp.float32) + projb_ref[...]
    logits = jnp.dot(h.astype(_BF), routwt_ref[...],
                     preferred_element_type=jnp.float32) + routb_ref[...]
    mx = jnp.max(logits, axis=-1, keepdims=True)
    e = jnp.exp(logits - mx)
    p = e / jnp.sum(e, axis=-1, keepdims=True)
    iota = jax.lax.broadcasted_iota(jnp.int32, (r, 8), 1)
    i1 = jnp.argmax(p, axis=-1)[:, None]
    m1 = iota == i1
    p_masked = jnp.where(m1, -1.0, p)
    i2 = jnp.argmax(p_masked, axis=-1)[:, None]
    m2 = iota == i2
    g = jnp.where(m1 | m2, p, 0.0)
    h_ref[...] = h
    g_ref[...] = g


def _moe_kernel(h_ref, g_ref, w1_ref, b1_ref, w2_ref, b2_ref,
                clswt_ref, clsb_ref, y_ref):
    h = h_ref[...].astype(_BF)
    g = g_ref[...]
    acc = jnp.zeros((_R2, MOE_DIM), dtype=jnp.float32)
    for j in range(E):
        hid = jnp.dot(h, w1_ref[j],
                      preferred_element_type=jnp.float32) + b1_ref[j][None, :]
        hid = jnp.maximum(hid, 0.0)
        eo = jnp.dot(hid.astype(_BF), w2_ref[j],
                     preferred_element_type=jnp.float32) + b2_ref[j][None, :]
        acc = acc + g[:, j:j + 1] * eo
    y_ref[...] = jnp.dot(acc.astype(_BF), clswt_ref[...],
                         preferred_element_type=jnp.float32) + clsb_ref[...]


def kernel(x, conv1_w, conv1_b, conv2_w, conv2_b, proj_w, proj_b,
           router_w, router_b, w1, b1, w2, b2, cls_w, cls_b):
    f32 = jnp.float32
    x = x.astype(f32)
    xpad = jnp.pad(x, ((0, 0), (6, 18)))
    xw = jnp.stack([xpad[:, 256 * i:256 * i + _KWIN] for i in range(_NW)],
                   axis=1).astype(_BF)
    wt = conv1_w[:, 0, :].T
    band = jnp.zeros((_KWIN, _WOUT, CONV_DIM), dtype=f32)
    for j in range(_WOUT):
        band = jax.lax.dynamic_update_slice(band, wt[:, None, :],
                                            (2 * j, j, 0))
    w1band = band.reshape(_KWIN, _WOUT * CONV_DIM).astype(_BF)
    b1big = jnp.tile(conv1_b, _WOUT)[None, :]
    import numpy as _np
    _i = _np.arange(_NW)[:, None, None]
    _blk = _np.arange(_WOUT // 8)[None, :, None]
    _l = _np.arange(8 * CONV_DIM)[None, None, :]
    _t = 128 * _i - 2 + 8 * _blk + _l // CONV_DIM
    mask = jnp.asarray(((_t >= 0) & (_t < 512)).astype(_np.float32))
    w2a = jnp.zeros((8 * CONV_DIM, 4 * CONV_DIM), dtype=f32)
    w2b = jnp.zeros((4 * CONV_DIM, 4 * CONV_DIM), dtype=f32)
    for delta in range(11):
        for rp in range(4):
            k = delta - 2 * rp
            if 0 <= k < 5:
                if delta < 8:
                    w2a = jax.lax.dynamic_update_slice(
                        w2a, conv2_w[:, :, k].T,
                        (delta * CONV_DIM, rp * CONV_DIM))
                else:
                    w2b = jax.lax.dynamic_update_slice(
                        w2b, conv2_w[:, :, k].T,
                        ((delta - 8) * CONV_DIM, rp * CONV_DIM))
    w2a = w2a.astype(_BF)
    w2b = w2b.astype(_BF)
    b2big = jnp.tile(conv2_b, 4)[None, :]
    projwt = proj_w.T.astype(_BF)
    projbr = proj_b[None, :]
    routwt = jnp.pad(router_w.T, ((0, 0), (0, 2))).astype(_BF)
    routb = jnp.pad(router_b, (0, 2), constant_values=-1e30)[None, :]

    grid1 = B // _R1
    h, g = pl.pallas_call(
        _frontend_kernel,
        grid=(grid1,),
        in_specs=[
            pl.BlockSpec((_R1, _NW, _KWIN), lambda i: (i, 0, 0)),
            pl.BlockSpec((_KWIN, _WOUT * CONV_DIM), lambda i: (0, 0)),
            pl.BlockSpec((1, _WOUT * CONV_DIM), lambda i: (0, 0)),
            pl.BlockSpec((_NW, _WOUT // 8, 8 * CONV_DIM),
                         lambda i: (0, 0, 0)),
            pl.BlockSpec((8 * CONV_DIM, 4 * CONV_DIM), lambda i: (0, 0)),
            pl.BlockSpec((4 * CONV_DIM, 4 * CONV_DIM), lambda i: (0, 0)),
            pl.BlockSpec((1, 4 * CONV_DIM), lambda i: (0, 0)),
            pl.BlockSpec((CONV_DIM, MOE_DIM), lambda i: (0, 0)),
            pl.BlockSpec((1, MOE_DIM), lambda i: (0, 0)),
            pl.BlockSpec((MOE_DIM, 8), lambda i: (0, 0)),
            pl.BlockSpec((1, 8), lambda i: (0, 0)),
        ],
        out_specs=[
            pl.BlockSpec((_R1, MOE_DIM), lambda i: (i, 0)),
            pl.BlockSpec((_R1, 8), lambda i: (i, 0)),
        ],
        out_shape=[
            jax.ShapeDtypeStruct((B, MOE_DIM), f32),
            jax.ShapeDtypeStruct((B, 8), f32),
        ],
    )(xw, w1band, b1big, mask, w2a, w2b, b2big, projwt, projbr, routwt,
      routb)

    grid2 = B // _R2
    y = pl.pallas_call(
        _moe_kernel,
        grid=(grid2,),
        in_specs=[
            pl.BlockSpec((_R2, MOE_DIM), lambda i: (i, 0)),
            pl.BlockSpec((_R2, 8), lambda i: (i, 0)),
            pl.BlockSpec((E, MOE_DIM, FF_DIM), lambda i: (0, 0, 0)),
            pl.BlockSpec((E, FF_DIM), lambda i: (0, 0)),
            pl.BlockSpec((E, FF_DIM, MOE_DIM), lambda i: (0, 0, 0)),
            pl.BlockSpec((E, MOE_DIM), lambda i: (0, 0)),
            pl.BlockSpec((MOE_DIM, NUM_CLASSES), lambda i: (0, 0)),
            pl.BlockSpec((1, NUM_CLASSES), lambda i: (0, 0)),
        ],
        out_specs=pl.BlockSpec((_R2, NUM_CLASSES), lambda i: (i, 0)),
        out_shape=jax.ShapeDtypeStruct((B, NUM_CLASSES), f32),
    )(h, g, w1.astype(_BF), b1, w2.astype(_BF), b2,
      cls_w.T.astype(_BF), cls_b[None, :])
    return y


# speedup vs baseline: 8.6588x; 3.2601x over previous
import numpy as np

import jax
import jax.numpy as jnp
from jax.experimental import pallas as pl

B, T = 1024, 1024
CONV_DIM, MOE_DIM, FF_DIM, E, NUM_CLASSES = 64, 512, 1024, 6, 10

_BF = jnp.bfloat16

_R1 = 64
_R2 = 256

_E1 = np.zeros((24, 8, 5), dtype=np.float32)
for _j in range(8):
    for _k in range(5):
        _E1[2 * _j + _k, _j, _k] = 1.0

_EA = np.zeros((8, 4, 5), dtype=np.float32)
for _q in range(8):
    for _r in range(4):
        _k = _q + 2 - 2 * _r
        if 0 <= _k < 5:
            _EA[_q, _r, _k] = 1.0
_EB = np.zeros((2, 4, 5), dtype=np.float32)
for _q in range(2):
    for _r in range(4):
        _k = _q - 2 * _r
        if 0 <= _k < 5:
            _EB[_q, _r, _k] = 1.0
_EC = np.zeros((1, 4, 5), dtype=np.float32)
for _r in range(4):
    _k = 10 - 2 * _r
    if 0 <= _k < 5:
        _EC[0, _r, _k] = 1.0


def _frontend_kernel(xw_ref, c1_ref, b1big_ref, w2f_ref, b2big_ref,
                     projwt_ref, projb_ref, routwt_ref, routb_ref,
                     h_ref, g_ref):
    r = _R1
    xwb = xw_ref[...].reshape(r * 64, 24)
    h1 = jnp.dot(xwb, c1_ref[...],
                 preferred_element_type=jnp.float32) + b1big_ref[...]
    a = jnp.maximum(h1, 0.0).astype(_BF).reshape(r, 64, 8 * CONV_DIM)
    zb = jnp.zeros((r, 1, 8 * CONV_DIM), dtype=_BF)
    prev = jnp.concatenate([zb, a[:, 0:63, :]], axis=1)[:, :, 6 * CONV_DIM:]
    nxt = jnp.concatenate([a[:, 1:64, :], zb], axis=1)[:, :, 0:CONV_DIM]
    lhs = jnp.concatenate([a, prev, nxt], axis=2)
    h2 = jnp.dot(lhs.reshape(r * 64, 11 * CONV_DIM), w2f_ref[...],
                 preferred_element_type=jnp.float32) + b2big_ref[...]
    h2 = jnp.maximum(h2, 0.0)
    h2s = jnp.sum(h2.reshape(r, 64, 4 * CONV_DIM), axis=1)
    m = (h2s[:, 0:64] + h2s[:, 64:128] + h2s[:, 128:192]
         + h2s[:, 192:256]) * (1.0 / 256.0)
    h = jnp.dot(m.astype(_BF), projwt_ref[...],
                preferred_element_type=jnp.float32) + projb_ref[...]
    logits = jnp.dot(h.astype(_BF), routwt_ref[...],
                     preferred_element_type=jnp.float32) + routb_ref[...]
    mx = jnp.max(logits, axis=-1, keepdims=True)
    e = jnp.exp(logits - mx)
    p = e / jnp.sum(e, axis=-1, keepdims=True)
    iota = jax.lax.broadcasted_iota(jnp.int32, (r, 8), 1)
    i1 = jnp.argmax(p, axis=-1)[:, None]
    m1 = iota == i1
    p_masked = jnp.where(m1, -1.0, p)
    i2 = jnp.argmax(p_masked, axis=-1)[:, None]
    m2 = iota == i2
    g = jnp.where(m1 | m2, p, 0.0)
    h_ref[...] = h
    g_ref[...] = g


def _moe_kernel(h_ref, g_ref, w1_ref, b1_ref, w2_ref, b2_ref,
                clswt_ref, clsb_ref, y_ref):
    h = h_ref[...].astype(_BF)
    g = g_ref[...]
    acc = jnp.zeros((_R2, MOE_DIM), dtype=jnp.float32)
    for j in range(E):
        hid = jnp.dot(h, w1_ref[j],
                      preferred_element_type=jnp.float32) + b1_ref[j][None, :]
        hid = jnp.maximum(hid, 0.0)
        eo = jnp.dot(hid.astype(_BF), w2_ref[j],
                     preferred_element_type=jnp.float32) + b2_ref[j][None, :]
        acc = acc + g[:, j:j + 1] * eo
    y_ref[...] = jnp.dot(acc.astype(_BF), clswt_ref[...],
                         preferred_element_type=jnp.float32) + clsb_ref[...]


def kernel(x, conv1_w, conv1_b, conv2_w, conv2_b, proj_w, proj_b,
           router_w, router_b, w1, b1, w2, b2, cls_w, cls_b):
    f32 = jnp.float32
    x = x.astype(f32)
    xpad = jnp.pad(x, ((0, 0), (2, 14)))
    a16 = xpad.reshape(B, 65, 16)
    xw = jnp.concatenate([a16[:, 0:64, :], a16[:, 1:65, 0:8]],
                         axis=2).astype(_BF)
    c1 = jnp.einsum('pjk,ck->pjc', jnp.asarray(_E1),
                    conv1_w[:, 0, :]).reshape(24, 8 * CONV_DIM).astype(_BF)
    b1big = jnp.tile(conv1_b, 8)[None, :]
    w2f = jnp.concatenate([
        jnp.einsum('qrk,dck->qcrd', jnp.asarray(_EA),
                   conv2_w).reshape(8 * CONV_DIM, 4 * CONV_DIM),
        jnp.einsum('qrk,dck->qcrd', jnp.asarray(_EB),
                   conv2_w).reshape(2 * CONV_DIM, 4 * CONV_DIM),
        jnp.einsum('qrk,dck->qcrd', jnp.asarray(_EC),
                   conv2_w).reshape(CONV_DIM, 4 * CONV_DIM),
    ], axis=0).astype(_BF)
    b2big = jnp.tile(conv2_b, 4)[None, :]
    projwt = proj_w.T.astype(_BF)
    projbr = proj_b[None, :]
    routwt = jnp.pad(router_w.T, ((0, 0), (0, 2))).astype(_BF)
    routb = jnp.pad(router_b, (0, 2), constant_values=-1e30)[None, :]

    grid1 = B // _R1
    h, g = pl.pallas_call(
        _frontend_kernel,
        grid=(grid1,),
        in_specs=[
            pl.BlockSpec((_R1, 64, 24), lambda i: (i, 0, 0)),
            pl.BlockSpec((24, 8 * CONV_DIM), lambda i: (0, 0)),
            pl.BlockSpec((1, 8 * CONV_DIM), lambda i: (0, 0)),
            pl.BlockSpec((11 * CONV_DIM, 4 * CONV_DIM), lambda i: (0, 0)),
            pl.BlockSpec((1, 4 * CONV_DIM), lambda i: (0, 0)),
            pl.BlockSpec((CONV_DIM, MOE_DIM), lambda i: (0, 0)),
            pl.BlockSpec((1, MOE_DIM), lambda i: (0, 0)),
            pl.BlockSpec((MOE_DIM, 8), lambda i: (0, 0)),
            pl.BlockSpec((1, 8), lambda i: (0, 0)),
        ],
        out_specs=[
            pl.BlockSpec((_R1, MOE_DIM), lambda i: (i, 0)),
            pl.BlockSpec((_R1, 8), lambda i: (i, 0)),
        ],
        out_shape=[
            jax.ShapeDtypeStruct((B, MOE_DIM), f32),
            jax.ShapeDtypeStruct((B, 8), f32),
        ],
    )(xw, c1, b1big, w2f, b2big, projwt, projbr, routwt, routb)

    grid2 = B // _R2
    y = pl.pallas_call(
        _moe_kernel,
        grid=(grid2,),
        in_specs=[
            pl.BlockSpec((_R2, MOE_DIM), lambda i: (i, 0)),
            pl.BlockSpec((_R2, 8), lambda i: (i, 0)),
            pl.BlockSpec((E, MOE_DIM, FF_DIM), lambda i: (0, 0, 0)),
            pl.BlockSpec((E, FF_DIM), lambda i: (0, 0)),
            pl.BlockSpec((E, FF_DIM, MOE_DIM), lambda i: (0, 0, 0)),
            pl.BlockSpec((E, MOE_DIM), lambda i: (0, 0)),
            pl.BlockSpec((MOE_DIM, NUM_CLASSES), lambda i: (0, 0)),
            pl.BlockSpec((1, NUM_CLASSES), lambda i: (0, 0)),
        ],
        out_specs=pl.BlockSpec((_R2, NUM_CLASSES), lambda i: (i, 0)),
        out_shape=jax.ShapeDtypeStruct((B, NUM_CLASSES), f32),
    )(h, g, w1.astype(_BF), b1, w2.astype(_BF), b2,
      cls_w.T.astype(_BF), cls_b[None, :])
    return y
